# baseline (speedup 1.0000x reference)
import functools

import jax
import jax.numpy as jnp
from jax import lax
from jax.experimental import pallas as pl
from jax.experimental.pallas import tpu as pltpu
from jax.experimental.pallas import tpu_sc as plsc

NN = 512
OPS = 32
E = NN * NN // 2
NSLOT = NN * NN
PAD = 2048
NC, NS, L = 2, 16, 16
NW = NC * NS
S = NSLOT // NW
CH = 16384
GB = 128
BLK = 2048


def _tc_sample_body(a_ref, u_ref, w_ref, l_ref):
    i = pl.program_id(0)
    a = a_ref[...]
    u = jnp.clip(u_ref[...], 1e-6, 1.0 - 1e-6)
    w = u / (u + (1.0 - u) * jnp.exp(-a))
    wp = w.reshape(OPS, BLK // 4, 4).transpose(1, 2, 0).reshape(BLK // 4, 4 * OPS)
    ap = a.reshape(OPS, BLK // 4, 4).transpose(1, 2, 0).reshape(BLK // 4, 4 * OPS)
    is_pad = i >= E // BLK
    w_ref[...] = jnp.where(is_pad, 0.0, wp)
    l_ref[...] = jnp.where(is_pad, 0.0, ap)


def _sample_and_pad(a_t, u_t):
    nblk = (E + PAD) // BLK
    last = E // BLK - 1
    return pl.pallas_call(
        _tc_sample_body,
        grid=(nblk,),
        in_specs=[pl.BlockSpec((OPS, BLK), lambda i: (0, jnp.minimum(i, last)))] * 2,
        out_specs=[pl.BlockSpec((BLK // 4, 4 * OPS), lambda i: (i, 0))] * 2,
        out_shape=[jax.ShapeDtypeStruct(((E + PAD) // 4, 4 * OPS), jnp.float32)] * 2,
    )(a_t, u_t)


def _sc_body(i0_hbm, i1_hbm, opw_hbm, alph_hbm, out_hbm, win, i0b, i1b, rows, sem):
    wid = lax.axis_index("s") * NC + lax.axis_index("c")
    base = wid * S
    iota = lax.broadcasted_iota(jnp.int32, (L,), 0)

    def init_row(j, carry):
        for k in range(GB // L):
            sent = E + ((j * GB + k * L + iota) & (PAD - 1))
            win[j, pl.ds(k * L, L)] = sent
        return carry

    lax.fori_loop(0, S // GB, init_row, 0)

    UNROLL = 8
    for c in range(E // CH):
        pltpu.sync_copy(i0_hbm.at[pl.ds(c * CH, CH)], i0b)
        pltpu.sync_copy(i1_hbm.at[pl.ds(c * CH, CH)], i1b)

        def scan_body(i, carry, c=c):
            for k in range(UNROLL):
                off = i * (UNROLL * L) + k * L
                v0 = i0b[pl.ds(off, L)]
                v1 = i1b[pl.ds(off, L)]
                rel = v0 * NN + v1 - base
                m = (rel >= 0) & (rel < S)
                relc = jnp.where(m, rel, 0)
                evec = (c * CH) + off + iota
                plsc.store_scatter(
                    win, [relc >> 7, relc & (GB - 1)], evec, mask=m)
            return carry

        lax.fori_loop(0, CH // (UNROLL * L), scan_body, 0)

    def emit(plane, src_hbm):
        def g_body(j, carry):
            pltpu.async_copy(src_hbm.at[win.at[j]], rows, sem).wait()
            pltpu.sync_copy(rows, out_hbm.at[plane, pl.ds(base + j * GB, GB)])
            return carry
        lax.fori_loop(0, S // GB, g_body, 0)

    emit(0, opw_hbm)
    emit(1, alph_hbm)


_sc_scatter = functools.partial(
    pl.kernel,
    out_type=jax.ShapeDtypeStruct((2, NSLOT, OPS), jnp.float32),
    mesh=plsc.VectorSubcoreMesh(core_axis_name="c", subcore_axis_name="s"),
    compiler_params=pltpu.CompilerParams(
        needs_layout_passes=False, use_tc_tiling_on_sc=False),
    scratch_types=[
        pltpu.VMEM((S // GB, GB), jnp.int32),
        pltpu.VMEM((CH,), jnp.int32),
        pltpu.VMEM((CH,), jnp.int32),
        pltpu.VMEM((GB, OPS), jnp.float32),
        pltpu.SemaphoreType.DMA,
    ],
)(_sc_body)


def _tc_relayout_body(a_ref, o_ref):
    for k in range(8):
        a = a_ref[0, pl.ds(k * 128, 128), :]
        o_ref[0, k] = a.reshape(128, 4, OPS).transpose(2, 0, 1).reshape(OPS, NN)


def _relayout(packed):
    return pl.pallas_call(
        _tc_relayout_body,
        grid=(2, NN // 8),
        in_specs=[pl.BlockSpec((1, 1024, 128), lambda p, d: (p, d, 0))],
        out_specs=pl.BlockSpec((1, 8, OPS, NN), lambda p, d: (p, d, 0, 0)),
        out_shape=jax.ShapeDtypeStruct((2, NN, OPS, NN), jnp.float32),
    )(packed)


def kernel(alphas, noise_u, idx):
    idx = idx.astype(jnp.int32)
    opw4, alph4 = _sample_and_pad(alphas.T, noise_u.T)
    opw_pad = opw4.reshape(E + PAD, OPS)
    alph_pad = alph4.reshape(E + PAD, OPS)
    out = _sc_scatter(idx[0], idx[1], opw_pad, alph_pad)
    out_t = _relayout(out.reshape(2, NSLOT * OPS // 128, 128))
    return jnp.transpose(out_t, (0, 1, 3, 2))


# speedup vs baseline: 3.2160x; 3.2160x over previous
import functools

import jax
import jax.numpy as jnp
from jax import lax
from jax.experimental import pallas as pl
from jax.experimental.pallas import tpu as pltpu
from jax.experimental.pallas import tpu_sc as plsc

NN = 512
OPS = 32
E = NN * NN // 2
NSLOT = NN * NN
PAD = 16384
EP = E + PAD
NC, NS, L = 2, 16, 16
NW = NC * NS
S = NSLOT // NW
DPW = S // NN
CH = 8192
GB = 128
TB = 2048
EPT = EP // NW
CH2 = EPT // 4


def _tc_sample_body(a_ref, u_ref, w_ref, l_ref):
    i = pl.program_id(0)
    a = a_ref[...]
    u = jnp.clip(u_ref[...], 1e-6, 1.0 - 1e-6)
    w = u / (u + (1.0 - u) * jnp.exp(-a))
    is_pad = i >= E // TB
    w_ref[...] = jnp.where(is_pad, 0.0, w)
    l_ref[...] = jnp.where(is_pad, 0.0, a)


def _sample(a_t, u_t):
    last = E // TB - 1
    return pl.pallas_call(
        _tc_sample_body,
        grid=(EP // TB,),
        in_specs=[pl.BlockSpec((OPS, TB), lambda i: (0, jnp.minimum(i, last)))] * 2,
        out_specs=[pl.BlockSpec((OPS, TB), lambda i: (0, i))] * 2,
        out_shape=[jax.ShapeDtypeStruct((OPS, EP), jnp.float32)] * 2,
    )(a_t, u_t)


def _sc_transpose_body(wt_hbm, at_hbm, we_hbm, ae_hbm, tin, tout, sem):
    wid = lax.axis_index("s") * NC + lax.axis_index("c")
    iota = lax.broadcasted_iota(jnp.int32, (L,), 0)
    iota32 = iota * OPS
    ccw = CH2 // GB

    for src_hbm, dst_hbm in ((wt_hbm, we_hbm), (at_hbm, ae_hbm)):
        for c in range(EPT // CH2):
            e0 = wid * EPT + c * CH2
            pltpu.sync_copy(src_hbm.at[:, pl.ds(e0 // GB, ccw), :, :], tin)

            def tr_body(i, carry):
                cc = i >> 3
                ec = i & 7
                for rb in range(4):
                    for r_in in range(8):
                        v = tin[rb, cc, r_in, pl.ds(ec * L, L)]
                        plsc.store_scatter(
                            tout,
                            [iota32 + ((cc * GB + ec * L) * OPS
                                       + rb * 8 + r_in)], v)
                return carry

            lax.fori_loop(0, CH2 // L, tr_body, 0)
            pltpu.sync_copy(tout, dst_hbm.at[pl.ds(e0 * OPS, CH2 * OPS)])


_sc_transpose = functools.partial(
    pl.kernel,
    out_type=(jax.ShapeDtypeStruct((EP * OPS,), jnp.float32),
              jax.ShapeDtypeStruct((EP * OPS,), jnp.float32)),
    mesh=plsc.VectorSubcoreMesh(core_axis_name="c", subcore_axis_name="s"),
    compiler_params=pltpu.CompilerParams(
        needs_layout_passes=False, use_tc_tiling_on_sc=False),
    scratch_types=[
        pltpu.VMEM((4, CH2 // GB, 8, GB), jnp.float32),
        pltpu.VMEM((CH2 * OPS,), jnp.float32),
        pltpu.SemaphoreType.DMA,
    ],
)(_sc_transpose_body)


def _sc_body(i0_hbm, i1_hbm, opw_hbm, alph_hbm, out_hbm,
             win, i0b0, i1b0, i0b1, i1b1, rows0, rows1, tb0, tb1,
             csem0, csem1, gsem0, gsem1, osem0, osem1):
    wid = lax.axis_index("s") * NC + lax.axis_index("c")
    base = wid * S
    dst0 = wid * DPW
    iota = lax.broadcasted_iota(jnp.int32, (L,), 0)

    def init_row(j, carry):
        for k in range(GB // L):
            sent = E + ((j * GB + k * L + iota) & (PAD - 1))
            win[j, pl.ds(k * L, L)] = sent
        return carry

    lax.fori_loop(0, S // GB, init_row, 0)

    i0b = (i0b0, i0b1)
    i1b = (i1b0, i1b1)
    csem = (csem0, csem1)
    UNROLL = 8
    NCHK = E // CH

    def issue_chunk(c, b):
        return (pltpu.async_copy(i0_hbm.at[pl.ds(c * CH, CH)], i0b[b], csem[b]),
                pltpu.async_copy(i1_hbm.at[pl.ds(c * CH, CH)], i1b[b], csem[b]))

    pend = issue_chunk(0, 0)
    for c in range(NCHK):
        b = c & 1
        cur = pend
        if c + 1 < NCHK:
            pend = issue_chunk(c + 1, 1 - b)
        cur[0].wait()
        cur[1].wait()

        def scan_body(i, carry, c=c, b=b):
            for k in range(UNROLL):
                off = i * (UNROLL * L) + k * L
                v0 = i0b[b][pl.ds(off, L)]
                v1 = i1b[b][pl.ds(off, L)]
                rel = v0 * NN + v1 - base
                m = (rel >= 0) & (rel < S)
                relc = jnp.where(m, rel, 0)
                evec = (c * CH) + off + iota
                plsc.store_scatter(
                    win, [relc >> 7, relc & (GB - 1)], evec, mask=m)
            return carry

        lax.fori_loop(0, CH // (UNROLL * L), scan_body, 0)

    rows = (rows0, rows1)
    tb = (tb0, tb1)
    gsem = (gsem0, gsem1)
    osem = (osem0, osem1)
    tvec0 = (iota >> 3) * (4 * 1024) + (iota & 7) * GB
    tvec1 = ((iota + L) >> 3) * (4 * 1024) + ((iota + L) & 7) * GB

    def issue_gather(src_hbm, d, b):
        return tuple(
            pltpu.async_copy(src_hbm.at[win.at[d * 4 + q]],
                             rows[b].at[pl.ds(q * GB, GB)], gsem[b])
            for q in range(4))

    units = [(p, s, d) for p, s in ((0, opw_hbm), (1, alph_hbm))
             for d in range(DPW)]
    wr = [None, None]
    gp = issue_gather(units[0][1], units[0][2], 0)
    for u, (plane, src_hbm, d) in enumerate(units):
        b = u & 1
        cur = gp
        if u + 1 < len(units):
            nxt = units[u + 1]
            gp = issue_gather(nxt[1], nxt[2], 1 - b)
        for dsc in cur:
            dsc.wait()
        if wr[b] is not None:
            wr[b].wait()

        def tr_body(s, carry, b=b):
            soff = (s >> 7) * 1024 + (s & (GB - 1))
            plsc.store_scatter(tb[b], [tvec0 + soff],
                               rows[b][s, pl.ds(0, L)])
            plsc.store_scatter(tb[b], [tvec1 + soff],
                               rows[b][s, pl.ds(L, L)])
            return carry

        lax.fori_loop(0, NN, tr_body, 0)
        wr[b] = pltpu.async_copy(tb[b], out_hbm.at[plane, dst0 + d],
                                 osem[b])
    wr[0].wait()
    wr[1].wait()


_sc_scatter = functools.partial(
    pl.kernel,
    out_type=jax.ShapeDtypeStruct((2, NN, OPS * NN), jnp.float32),
    mesh=plsc.VectorSubcoreMesh(core_axis_name="c", subcore_axis_name="s"),
    compiler_params=pltpu.CompilerParams(
        needs_layout_passes=False, use_tc_tiling_on_sc=False),
    scratch_types=[
        pltpu.VMEM((S // GB, GB), jnp.int32),
        pltpu.VMEM((CH,), jnp.int32),
        pltpu.VMEM((CH,), jnp.int32),
        pltpu.VMEM((CH,), jnp.int32),
        pltpu.VMEM((CH,), jnp.int32),
        pltpu.VMEM((NN, OPS), jnp.float32),
        pltpu.VMEM((NN, OPS), jnp.float32),
        pltpu.VMEM((OPS * NN,), jnp.float32),
        pltpu.VMEM((OPS * NN,), jnp.float32),
        pltpu.SemaphoreType.DMA,
        pltpu.SemaphoreType.DMA,
        pltpu.SemaphoreType.DMA,
        pltpu.SemaphoreType.DMA,
        pltpu.SemaphoreType.DMA,
        pltpu.SemaphoreType.DMA,
    ],
)(_sc_body)


def kernel(alphas, noise_u, idx):
    idx = idx.astype(jnp.int32)
    w_t, a_t = _sample(alphas.T, noise_u.T)
    w_t4 = w_t.reshape(4, 8, EP // GB, GB).transpose(0, 2, 1, 3)
    a_t4 = a_t.reshape(4, 8, EP // GB, GB).transpose(0, 2, 1, 3)
    w_e, a_e = _sc_transpose(w_t4, a_t4)
    out = _sc_scatter(idx[0], idx[1],
                      w_e.reshape(EP, OPS), a_e.reshape(EP, OPS))
    out6 = out.reshape(2, NN, 4, 4, 8, GB)
    return out6.transpose(0, 1, 3, 5, 2, 4).reshape(2, NN, NN, OPS)
